# Initial kernel scaffold; baseline (speedup 1.0000x reference)
import jax
import jax.numpy as jnp
from jax.experimental import pallas as pl


def kernel(vertices, E_mask, embed_map):
    raise NotImplementedError("write your pallas kernel here")


# speedup vs baseline: 1.6261x; 1.6261x over previous
import functools

import jax
import jax.numpy as jnp
from jax import lax
from jax.experimental import pallas as pl
from jax.experimental.pallas import tpu as pltpu
from jax.experimental.pallas import tpu_sc as plsc

VOCAB = 100000
D = 64
B, H, W = 4, 384, 384
P = H * W
N = B * P
LANES = 16

SUP = 1024
C = 512
CG = C // 128


def _make_kernel():
    info = plsc.get_sparse_core_info()
    NC, NS = info.num_cores, info.num_subcores
    NW = NC * NS
    per_w = N // NW
    assert N % NW == 0 and per_w % SUP == 0 and P % per_w == 0
    n_sup = per_w // SUP

    mesh = plsc.VectorSubcoreMesh(core_axis_name="c", subcore_axis_name="s")

    @functools.partial(
        pl.kernel,
        mesh=mesh,
        compiler_params=pltpu.CompilerParams(
            needs_layout_passes=False,
            use_tc_tiling_on_sc=False,
        ),
        out_type=jax.ShapeDtypeStruct((B * D, P), jnp.float32),
        scratch_types=[
            pltpu.VMEM((SUP // 128, 128), jnp.int32),
            pltpu.VMEM((SUP,), jnp.float32),
            pltpu.VMEM((C, D), jnp.float32),
            pltpu.VMEM((D, C), jnp.float32),
            pltpu.SemaphoreType.DMA,
        ],
    )
    def k(idx_hbm, mask_hbm, table_hbm, out_hbm, idx_v, mask_v, rows_v, trans_v, sem):
        wid = lax.axis_index("s") * NC + lax.axis_index("c")
        base = wid * per_w
        b = base // P
        p0 = base - b * P
        bd0 = b * D

        def sup_body(s, carry):
            start = base + s * SUP
            irow = pl.multiple_of(start // 128, 8)
            pltpu.sync_copy(idx_hbm.at[pl.ds(irow, SUP // 128)], idx_v)
            pltpu.sync_copy(mask_hbm.at[pl.ds(pl.multiple_of(start, SUP), SUP)], mask_v)

            for h in range(SUP // C):
                cps = [
                    pltpu.async_copy(
                        table_hbm.at[idx_v.at[h * CG + j]],
                        rows_v.at[pl.ds(j * 128, 128)],
                        sem,
                    )
                    for j in range(CG)
                ]
                for cp in cps:
                    cp.wait()

                def g_body(g, c2, h=h):
                    g16 = g * LANES
                    mv = mask_v[pl.ds(h * C + g16, LANES)]
                    p_vec = g16 + lax.iota(jnp.int32, LANES)
                    for d in range(D):
                        d_vec = jnp.full((LANES,), d, jnp.int32)
                        vals = plsc.load_gather(rows_v, [p_vec, d_vec])
                        trans_v[d, pl.ds(g16, LANES)] = vals * mv
                    return c2

                lax.fori_loop(0, C // LANES, g_body, 0)

                pltpu.sync_copy(
                    trans_v,
                    out_hbm.at[
                        pl.ds(pl.multiple_of(bd0, D), D),
                        pl.ds(pl.multiple_of(p0 + s * SUP + h * C, C), C),
                    ],
                )
            return carry

        lax.fori_loop(0, n_sup, sup_body, 0)

    return k


_kernel = _make_kernel()


@jax.jit
def kernel(vertices, E_mask, embed_map):
    idx2d = vertices.reshape(-1).astype(jnp.int32).reshape(N // 128, 128)
    maskf = E_mask.reshape(-1)
    out2d = _kernel(idx2d, maskf, embed_map)
    return out2d.reshape(B, D, H, W)


# speedup vs baseline: 2.8449x; 1.7495x over previous
import functools

import jax
import jax.numpy as jnp
from jax import lax
from jax.experimental import pallas as pl
from jax.experimental.pallas import tpu as pltpu
from jax.experimental.pallas import tpu_sc as plsc

VOCAB = 100000
D = 64
B, H, W = 4, 384, 384
P = H * W
N = B * P
LANES = 16

C = 256
CG = C // 128
NBUF = 3
OBUF = 2
DB = 8


def _make_kernel():
    info = plsc.get_sparse_core_info()
    NC, NS = info.num_cores, info.num_subcores
    NW = NC * NS
    per_w = N // NW
    assert N % NW == 0 and per_w % C == 0 and P % per_w == 0
    n_chunks = per_w // C
    irows = per_w // 128
    unroll = NBUF * OBUF
    assert n_chunks % unroll == 0

    mesh = plsc.VectorSubcoreMesh(core_axis_name="c", subcore_axis_name="s")

    @functools.partial(
        pl.kernel,
        mesh=mesh,
        compiler_params=pltpu.CompilerParams(
            needs_layout_passes=False,
            use_tc_tiling_on_sc=False,
        ),
        out_type=jax.ShapeDtypeStruct((B * D, P), jnp.float32),
        scratch_types=[
            pltpu.VMEM((irows, 128), jnp.int32),
            pltpu.VMEM((per_w,), jnp.float32),
            [pltpu.VMEM((C, D), jnp.float32)] * NBUF,
            [pltpu.VMEM((D, C), jnp.float32)] * OBUF,
            [pltpu.SemaphoreType.DMA] * NBUF,
            [pltpu.SemaphoreType.DMA] * OBUF,
        ],
    )
    def k(idx_hbm, mask_hbm, table_hbm, out_hbm, idx_v, mask_v, rows, trans,
          gsem, osem):
        wid = lax.axis_index("s") * NC + lax.axis_index("c")
        base = wid * per_w
        b = base // P
        p0 = base - b * P
        bd0 = b * D
        iota = lax.iota(jnp.int32, LANES)

        pltpu.sync_copy(idx_hbm.at[pl.ds(pl.multiple_of(wid * irows, 8), irows)],
                        idx_v)
        pltpu.sync_copy(mask_hbm.at[pl.ds(pl.multiple_of(base, per_w), per_w)],
                        mask_v)

        def gather(c, buf):
            return [
                pltpu.make_async_copy(
                    table_hbm.at[idx_v.at[c * CG + j]],
                    rows[buf].at[pl.ds(j * 128, 128)],
                    gsem[buf],
                )
                for j in range(CG)
            ]

        def out_copy(c, tb):
            return pltpu.make_async_copy(
                trans[tb],
                out_hbm.at[
                    pl.ds(pl.multiple_of(bd0, D), D),
                    pl.ds(pl.multiple_of(p0 + c * C, C), C),
                ],
                osem[tb],
            )

        for buf in range(NBUF):
            for cp in gather(buf, buf):
                cp.start()

        def super_body(s, carry):
            for ks in range(unroll):
                i = s * unroll + ks
                buf = ks % NBUF
                tb = ks % OBUF
                for cp in gather(i, buf):
                    cp.wait()

                @pl.when(i >= OBUF)
                def _():
                    out_copy(i - OBUF, tb).wait()

                def g_body(g, c2, buf=buf, tb=tb):
                    g16 = g * LANES
                    mv = mask_v[pl.ds(i * C + g16, LANES)]
                    p_vec = g16 + iota
                    for d0 in range(0, D, DB):
                        vs = [
                            plsc.load_gather(
                                rows[buf],
                                [p_vec, jnp.full((LANES,), d, jnp.int32)],
                            )
                            for d in range(d0, d0 + DB)
                        ]
                        for kk, d in enumerate(range(d0, d0 + DB)):
                            trans[tb][d, pl.ds(g16, LANES)] = vs[kk] * mv
                    return c2

                lax.fori_loop(0, C // LANES, g_body, 0)

                @pl.when(i + NBUF < n_chunks)
                def _():
                    for cp in gather(i + NBUF, buf):
                        cp.start()

                out_copy(i, tb).start()
            return carry

        lax.fori_loop(0, n_chunks // unroll, super_body, 0)

        for tail in range(OBUF):
            c = n_chunks - OBUF + tail
            out_copy(c, c % OBUF).wait()

    return k


_kernel = _make_kernel()


@jax.jit
def kernel(vertices, E_mask, embed_map):
    idx2d = vertices.reshape(-1).astype(jnp.int32).reshape(N // 128, 128)
    maskf = E_mask.reshape(-1)
    out2d = _kernel(idx2d, maskf, embed_map)
    return out2d.reshape(B, D, H, W)


# speedup vs baseline: 3.6538x; 1.2843x over previous
import functools

import jax
import jax.numpy as jnp
from jax import lax
from jax.experimental import pallas as pl
from jax.experimental.pallas import tpu as pltpu
from jax.experimental.pallas import tpu_sc as plsc

VOCAB = 100000
D = 64
B, H, W = 4, 384, 384
P = H * W
N = B * P
LANES = 16

C = 256
CS = C + 1
CG = C // 128
NBUF = 3
OBUF = 2


def _make_kernel():
    info = plsc.get_sparse_core_info()
    NC, NS = info.num_cores, info.num_subcores
    NW = NC * NS
    per_w = N // NW
    assert N % NW == 0 and per_w % C == 0 and P % per_w == 0
    n_chunks = per_w // C
    irows = per_w // 128
    unroll = NBUF * OBUF
    assert n_chunks % unroll == 0

    mesh = plsc.VectorSubcoreMesh(core_axis_name="c", subcore_axis_name="s")

    @functools.partial(
        pl.kernel,
        mesh=mesh,
        compiler_params=pltpu.CompilerParams(
            needs_layout_passes=False,
            use_tc_tiling_on_sc=False,
        ),
        out_type=jax.ShapeDtypeStruct((B * D, P), jnp.float32),
        scratch_types=[
            pltpu.VMEM((irows, 128), jnp.int32),
            pltpu.VMEM((per_w,), jnp.float32),
            [pltpu.VMEM((C, D), jnp.float32)] * NBUF,
            [pltpu.VMEM((D, CS), jnp.float32)] * OBUF,
            [pltpu.SemaphoreType.DMA] * NBUF,
            [pltpu.SemaphoreType.DMA] * OBUF,
        ],
    )
    def k(idx_hbm, mask_hbm, table_hbm, out_hbm, idx_v, mask_v, rows, trans,
          gsem, osem):
        wid = lax.axis_index("s") * NC + lax.axis_index("c")
        base = wid * per_w
        b = base // P
        p0 = base - b * P
        bd0 = b * D
        iota = lax.iota(jnp.int32, LANES)
        d_vecs = [kq * LANES + iota for kq in range(D // LANES)]

        pltpu.sync_copy(idx_hbm.at[pl.ds(pl.multiple_of(wid * irows, 8), irows)],
                        idx_v)
        pltpu.sync_copy(mask_hbm.at[pl.ds(pl.multiple_of(base, per_w), per_w)],
                        mask_v)

        def gather(c, buf):
            return [
                pltpu.make_async_copy(
                    table_hbm.at[idx_v.at[c * CG + j]],
                    rows[buf].at[pl.ds(j * 128, 128)],
                    gsem[buf],
                )
                for j in range(CG)
            ]

        def out_copy(c, tb):
            return pltpu.make_async_copy(
                trans[tb].at[:, pl.ds(0, C)],
                out_hbm.at[
                    pl.ds(pl.multiple_of(bd0, D), D),
                    pl.ds(pl.multiple_of(p0 + c * C, C), C),
                ],
                osem[tb],
            )

        for buf in range(NBUF):
            for cp in gather(buf, buf):
                cp.start()

        def super_body(s, carry):
            for ks in range(unroll):
                i = s * unroll + ks
                buf = ks % NBUF
                tb = ks % OBUF
                for cp in gather(i, buf):
                    cp.wait()

                @pl.when(i >= OBUF)
                def _():
                    out_copy(i - OBUF, tb).wait()

                def g_body(g, c2, buf=buf, tb=tb):
                    g16 = g * LANES
                    mvec = mask_v[pl.ds(i * C + g16, LANES)]
                    for p16 in range(LANES):
                        p = g16 + p16
                        mb = jnp.full((LANES,), mvec[p16])
                        p_vec = jnp.full((LANES,), p, jnp.int32)
                        for kq in range(D // LANES):
                            v = rows[buf][p, pl.ds(kq * LANES, LANES)]
                            plsc.store_scatter(
                                trans[tb], [d_vecs[kq], p_vec], v * mb
                            )
                    return c2

                lax.fori_loop(0, C // LANES, g_body, 0)

                @pl.when(i + NBUF < n_chunks)
                def _():
                    for cp in gather(i + NBUF, buf):
                        cp.start()

                out_copy(i, tb).start()
            return carry

        lax.fori_loop(0, n_chunks // unroll, super_body, 0)

        for tail in range(OBUF):
            c = n_chunks - OBUF + tail
            out_copy(c, c % OBUF).wait()

    return k


_kernel = _make_kernel()


@jax.jit
def kernel(vertices, E_mask, embed_map):
    idx2d = vertices.reshape(-1).astype(jnp.int32).reshape(N // 128, 128)
    maskf = E_mask.reshape(-1)
    out2d = _kernel(idx2d, maskf, embed_map)
    return out2d.reshape(B, D, H, W)


# speedup vs baseline: 3.6632x; 1.0026x over previous
import functools

import jax
import jax.numpy as jnp
from jax import lax
from jax.experimental import pallas as pl
from jax.experimental.pallas import tpu as pltpu
from jax.experimental.pallas import tpu_sc as plsc

VOCAB = 100000
D = 64
B, H, W = 4, 384, 384
P = H * W
N = B * P
LANES = 16

C = W
CS = C + 1
CG = C // 128
NBUF = 3
OBUF = 2
ISLOT = 6
MSLOT = 2


def _make_kernel():
    info = plsc.get_sparse_core_info()
    NC, NS = info.num_cores, info.num_subcores
    NW = NC * NS
    per_w = N // NW
    rows_w = per_w // C
    assert N % NW == 0 and per_w % C == 0 and P % per_w == 0
    unroll = ISLOT
    assert rows_w % unroll == 0

    mesh = plsc.VectorSubcoreMesh(core_axis_name="c", subcore_axis_name="s")

    @functools.partial(
        pl.kernel,
        mesh=mesh,
        compiler_params=pltpu.CompilerParams(
            needs_layout_passes=False,
            use_tc_tiling_on_sc=False,
        ),
        out_type=jax.ShapeDtypeStruct((B, D, H, W), jnp.float32),
        scratch_types=[
            pltpu.VMEM((ISLOT * C,), jnp.int32),
            pltpu.VMEM((MSLOT * C,), jnp.float32),
            [pltpu.VMEM((C, D), jnp.float32)] * NBUF,
            [pltpu.VMEM((D, CS), jnp.float32)] * OBUF,
            [pltpu.SemaphoreType.DMA] * NBUF,
            [pltpu.SemaphoreType.DMA] * OBUF,
            [pltpu.SemaphoreType.DMA] * ISLOT,
            [pltpu.SemaphoreType.DMA] * MSLOT,
        ],
    )
    def k(idx_hbm, mask_hbm, table_hbm, out_hbm, idx_v, mask_v, rows, trans,
          gsem, osem, isem, msem):
        wid = lax.axis_index("s") * NC + lax.axis_index("c")
        wpi = P // per_w
        b = wid // wpi
        r0 = (wid - b * wpi) * rows_w
        iota = lax.iota(jnp.int32, LANES)
        d_vecs = [kq * LANES + iota for kq in range(D // LANES)]

        def idx_copy(c, sl):
            return pltpu.make_async_copy(
                idx_hbm.at[b, 0, r0 + c, :],
                idx_v.at[pl.ds(sl * C, C)],
                isem[sl],
            )

        def mask_copy(c, sl):
            return pltpu.make_async_copy(
                mask_hbm.at[b, 0, r0 + c, :],
                mask_v.at[pl.ds(sl * C, C)],
                msem[sl],
            )

        def gather(sl_i, buf):
            return [
                pltpu.make_async_copy(
                    table_hbm.at[idx_v.at[pl.ds(sl_i * C + j * 128, 128)]],
                    rows[buf].at[pl.ds(j * 128, 128)],
                    gsem[buf],
                )
                for j in range(CG)
            ]

        def out_copy(c, tb):
            return pltpu.make_async_copy(
                trans[tb].at[:, pl.ds(0, C)],
                out_hbm.at[b, :, r0 + c, :],
                osem[tb],
            )

        for sl in range(ISLOT):
            idx_copy(sl, sl).start()
        for sl in range(MSLOT):
            mask_copy(sl, sl).start()
        for c in range(NBUF):
            idx_copy(c, c).wait()
            for cp in gather(c, c):
                cp.start()

        def super_body(s, carry):
            for ks in range(unroll):
                i = s * unroll + ks
                buf = ks % NBUF
                tb = ks % OBUF
                msl = ks % MSLOT
                for cp in gather(ks % ISLOT, buf):
                    cp.wait()
                mask_copy(i, msl).wait()

                @pl.when(i + ISLOT < rows_w)
                def _():
                    idx_copy(i + ISLOT, ks).start()

                @pl.when(i >= OBUF)
                def _():
                    out_copy(i - OBUF, tb).wait()

                def g_body(g, c2, buf=buf, tb=tb, msl=msl):
                    g16 = g * LANES
                    mvec = mask_v[pl.ds(msl * C + g16, LANES)]
                    for p16 in range(LANES):
                        p = g16 + p16
                        mb = jnp.full((LANES,), mvec[p16])
                        p_vec = jnp.full((LANES,), p, jnp.int32)
                        for kq in range(D // LANES):
                            v = rows[buf][p, pl.ds(kq * LANES, LANES)]
                            plsc.store_scatter(
                                trans[tb], [d_vecs[kq], p_vec], v * mb
                            )
                    return c2

                lax.fori_loop(0, C // LANES, g_body, 0)

                @pl.when(i + NBUF < rows_w)
                def _():
                    idx_copy(i + NBUF, (ks + NBUF) % ISLOT).wait()
                    for cp in gather((ks + NBUF) % ISLOT, buf):
                        cp.start()

                @pl.when(i + MSLOT < rows_w)
                def _():
                    mask_copy(i + MSLOT, msl).start()

                out_copy(i, tb).start()
            return carry

        lax.fori_loop(0, rows_w // unroll, super_body, 0)

        for tail in range(OBUF):
            c = rows_w - OBUF + tail
            out_copy(c, c % OBUF).wait()

    return k


_kernel = _make_kernel()


@jax.jit
def kernel(vertices, E_mask, embed_map):
    return _kernel(vertices.astype(jnp.int32), E_mask, embed_map)
